# Initial kernel scaffold; baseline (speedup 1.0000x reference)
import jax
import jax.numpy as jnp
from jax.experimental import pallas as pl


def kernel(x, edge_index, batch, W1, b1, W2, b2, W3, b3, Wg, bg, Wn1, bn1, Wn2, bn2):
    raise NotImplementedError("write your pallas kernel here")


# speedup vs baseline: 12.3020x; 12.3020x over previous
import functools

import jax
import jax.numpy as jnp
from jax import lax
from jax.experimental import pallas as pl
from jax.experimental.pallas import tpu as pltpu
from jax.experimental.pallas import tpu_sc as plsc

f32 = jnp.float32

NC = 2
NS = 16
NW = NC * NS
LANES = 16
CHUNK = 128


def _mesh():
  return plsc.VectorSubcoreMesh(
      core_axis_name="c", subcore_axis_name="s", num_cores=NC, num_subcores=NS)


def _sc_degree(dst2d, npad, cpw):
  nrows = npad // CHUNK

  @functools.partial(
      pl.kernel,
      out_type=jax.ShapeDtypeStruct((NC, nrows, CHUNK), f32),
      mesh=_mesh(),
      compiler_params=pltpu.CompilerParams(needs_layout_passes=False),
      scratch_types=[
          pltpu.VMEM((cpw, CHUNK), jnp.int32),
          pltpu.VMEM((npad,), f32),
          pltpu.VMEM((nrows, CHUNK), f32),
          pltpu.VMEM((nrows,), jnp.int32),
          pltpu.VMEM_SHARED((nrows, CHUNK), f32),
          pltpu.SemaphoreType.DMA,
      ],
  )
  def deg_kernel(dst_hbm, out_hbm, idx_v, deg_v, deg2_v, ident_v, deg_sh,
                 sem):
    cid = lax.axis_index("c")
    sid = lax.axis_index("s")
    wid = cid * NS + sid

    zero = jnp.zeros((LANES,), f32)

    def zbody(i, carry):
      for k in range(CHUNK // LANES):
        deg2_v[i, pl.ds(k * LANES, LANES)] = zero
        deg_v[pl.ds(i * CHUNK + k * LANES, LANES)] = zero
      return carry
    lax.fori_loop(0, nrows, zbody, 0)

    @pl.when(sid == 0)
    def _():
      pltpu.sync_copy(deg2_v, deg_sh)

    def ibody(k, carry):
      ident_v[pl.ds(k * LANES, LANES)] = (
          lax.iota(jnp.int32, LANES) + k * LANES)
      return carry
    lax.fori_loop(0, nrows // LANES, ibody, 0)

    pltpu.sync_copy(dst_hbm.at[pl.ds(wid * cpw, cpw)], idx_v)

    ones = jnp.ones((LANES,), f32)

    def ebody(i, carry):
      for k in range(CHUNK // LANES):
        d = idx_v[i, pl.ds(k * LANES, LANES)]
        plsc.addupdate_scatter(deg_v, [d], ones)
      return carry
    lax.fori_loop(0, cpw, ebody, 0)

    def sbody(i, carry):
      for k in range(CHUNK // LANES):
        deg2_v[i, pl.ds(k * LANES, LANES)] = (
            deg_v[pl.ds(i * CHUNK + k * LANES, LANES)])
      return carry
    lax.fori_loop(0, nrows, sbody, 0)

    plsc.subcore_barrier()
    pltpu.sync_copy(deg2_v, deg_sh.at[ident_v], add=True)
    plsc.subcore_barrier()

    @pl.when(sid < nrows // 8)
    def _():
      base = sid * 8
      pltpu.sync_copy(deg_sh.at[pl.ds(base, 8)], deg2_v.at[pl.ds(0, 8)])
      pltpu.sync_copy(deg2_v.at[pl.ds(0, 8)], out_hbm.at[cid, pl.ds(base, 8)])

  return deg_kernel


def _sc_propagate(src2d, dst2d, u, npad, cpw, h):
  rows_per_tile = npad // NS
  assert rows_per_tile % CHUNK == 0 or rows_per_tile < CHUNK
  ncopy = rows_per_tile // CHUNK

  @functools.partial(
      pl.kernel,
      out_type=jax.ShapeDtypeStruct((NC, npad, h), f32),
      mesh=_mesh(),
      compiler_params=pltpu.CompilerParams(needs_layout_passes=False,
                                           use_tc_tiling_on_sc=False),
      scratch_types=[
          pltpu.VMEM((cpw, CHUNK), jnp.int32),
          pltpu.VMEM((cpw, CHUNK), jnp.int32),
          pltpu.VMEM((CHUNK, h), f32),
          pltpu.VMEM_SHARED((npad, h), f32),
          pltpu.SemaphoreType.DMA,
      ],
  )
  def prop_kernel(src_hbm, dst_hbm, u_hbm, out_hbm, src_v, dst_v, rows_v,
                  acc_sh, sem):
    cid = lax.axis_index("c")
    sid = lax.axis_index("s")
    wid = cid * NS + sid

    zero = jnp.zeros((LANES,), f32)

    def zbody(i, carry):
      for k in range(h // LANES):
        rows_v[i, pl.ds(k * LANES, LANES)] = zero
      return carry
    lax.fori_loop(0, CHUNK, zbody, 0)

    base = sid * rows_per_tile
    for k in range(ncopy):
      pltpu.sync_copy(rows_v, acc_sh.at[pl.ds(base + k * CHUNK, CHUNK)])

    pltpu.sync_copy(src_hbm.at[pl.ds(wid * cpw, cpw)], src_v)
    pltpu.sync_copy(dst_hbm.at[pl.ds(wid * cpw, cpw)], dst_v)

    plsc.subcore_barrier()

    def chunk_body(j, carry):
      pltpu.async_copy(u_hbm.at[src_v.at[j]], rows_v, sem).wait()
      pltpu.sync_copy(rows_v, acc_sh.at[dst_v.at[j]], add=True)
      return carry
    lax.fori_loop(0, cpw, chunk_body, 0)

    plsc.subcore_barrier()

    for k in range(ncopy):
      pltpu.sync_copy(acc_sh.at[pl.ds(base + k * CHUNK, CHUNK)], rows_v)
      pltpu.sync_copy(rows_v, out_hbm.at[cid, pl.ds(base + k * CHUNK, CHUNK)])

  return prop_kernel(src2d, dst2d, u)


def _tc_pre(degsum, x, W1, n, h):
  def body(deg_ref, x_ref, w_ref, u_ref, dis_ref):
    d = jnp.maximum(deg_ref[...], 1.0)
    dis = lax.rsqrt(d)
    dis = dis * (1.5 - 0.5 * d * dis * dis)
    z = jnp.dot(x_ref[...], w_ref[...], preferred_element_type=f32,
                precision=lax.Precision.HIGHEST)
    u_ref[...] = dis * z
    dis_ref[...] = dis

  return pl.pallas_call(
      body,
      out_shape=(jax.ShapeDtypeStruct((n, h), f32),
                 jax.ShapeDtypeStruct((n, 1), f32)),
  )(degsum, x, W1)


def _tc_mid(S, u, dis, b, W, n, h):
  def body(S_ref, u_ref, dis_ref, b_ref, w_ref, out_ref):
    s = S_ref[0, :n, :] + S_ref[1, :n, :] + u_ref[...]
    hh = jnp.maximum(dis_ref[...] * s + b_ref[...], 0.0)
    out_ref[...] = dis_ref[...] * jnp.dot(
        hh, w_ref[...], preferred_element_type=f32,
        precision=lax.Precision.HIGHEST)

  return pl.pallas_call(
      body,
      out_shape=jax.ShapeDtypeStruct((n, h), f32),
  )(S, u, dis, b, W)


def _tc_final(S, u, dis, b3, Wg, bg, Wn1, bn1, Wn2, bn2, n, h):
  def body(S_ref, u_ref, dis_ref, b_ref, wg_ref, bg_ref, wn1_ref, bn1_ref,
           wn2_ref, bn2_ref, glog_ref, nlog_ref):
    s = S_ref[0, :n, :] + S_ref[1, :n, :] + u_ref[...]
    emb = jnp.maximum(dis_ref[...] * s + b_ref[...], 0.0)
    pooled = jnp.sum(emb, axis=0, keepdims=True) * (1.0 / n)
    glog_ref[...] = jnp.dot(pooled, wg_ref[...], preferred_element_type=f32,
                            precision=lax.Precision.HIGHEST) + bg_ref[...]
    nh = jnp.maximum(
        jnp.dot(emb, wn1_ref[...], preferred_element_type=f32,
                precision=lax.Precision.HIGHEST) + bn1_ref[...], 0.0)
    nlog_ref[...] = jnp.dot(nh, wn2_ref[...], preferred_element_type=f32,
                            precision=lax.Precision.HIGHEST) + bn2_ref[...]

  return pl.pallas_call(
      body,
      out_shape=(jax.ShapeDtypeStruct((1, 2), f32),
                 jax.ShapeDtypeStruct((n, 2), f32)),
  )(S, u, dis, b3, Wg, bg, Wn1, bn1, Wn2, bn2)


def kernel(x, edge_index, batch, W1, b1, W2, b2, W3, b3, Wg, bg, Wn1, bn1,
           Wn2, bn2):
  n, f_in = x.shape
  h = W1.shape[1]
  e = edge_index.shape[1]

  npad = -(-n // (NS * CHUNK)) * (NS * CHUNK)
  cpw = -(-e // (NW * CHUNK))
  cpw = -(-cpw // 8) * 8
  epad = NW * cpw * CHUNK

  trash = npad - 8
  src_p = jnp.concatenate(
      [edge_index[0], jnp.zeros((epad - e,), jnp.int32)]).reshape(-1, CHUNK)
  dst_p = jnp.concatenate(
      [edge_index[1], jnp.full((epad - e,), trash, jnp.int32)]).reshape(
          -1, CHUNK)

  degp = _sc_degree(dst_p, npad, cpw)(dst_p)
  degsum = (degp[0] + degp[1]).reshape(npad, 1)[:n] + 1.0

  u1, dis = _tc_pre(degsum, x, W1, n, h)
  S1 = _sc_propagate(src_p, dst_p, u1, npad, cpw, h)
  u2 = _tc_mid(S1, u1, dis, b1.reshape(1, h), W2, n, h)
  S2 = _sc_propagate(src_p, dst_p, u2, npad, cpw, h)
  u3 = _tc_mid(S2, u2, dis, b2.reshape(1, h), W3, n, h)
  S3 = _sc_propagate(src_p, dst_p, u3, npad, cpw, h)

  return _tc_final(S3, u3, dis, b3.reshape(1, h), Wg, bg.reshape(1, 2),
                   Wn1, bn1.reshape(1, h // 2), Wn2, bn2.reshape(1, 2), n, h)


# speedup vs baseline: 14.6774x; 1.1931x over previous
import functools

import jax
import jax.numpy as jnp
from jax import lax
from jax.experimental import pallas as pl
from jax.experimental.pallas import tpu as pltpu
from jax.experimental.pallas import tpu_sc as plsc

f32 = jnp.float32

NC = 2
NS = 16
NW = NC * NS
LANES = 16
CHUNK = 128
NBUF = 4


def _mesh():
  return plsc.VectorSubcoreMesh(
      core_axis_name="c", subcore_axis_name="s", num_cores=NC, num_subcores=NS)


def _sc_degree(dst2d, npad, cpw):
  nrows = npad // CHUNK

  @functools.partial(
      pl.kernel,
      out_type=jax.ShapeDtypeStruct((NC, nrows, CHUNK), f32),
      mesh=_mesh(),
      compiler_params=pltpu.CompilerParams(needs_layout_passes=False),
      scratch_types=[
          pltpu.VMEM((cpw, CHUNK), jnp.int32),
          pltpu.VMEM((npad,), f32),
          pltpu.VMEM((nrows, CHUNK), f32),
          pltpu.VMEM((nrows,), jnp.int32),
          pltpu.VMEM_SHARED((nrows, CHUNK), f32),
          pltpu.SemaphoreType.DMA,
      ],
  )
  def deg_kernel(dst_hbm, out_hbm, idx_v, deg_v, deg2_v, ident_v, deg_sh,
                 sem):
    cid = lax.axis_index("c")
    sid = lax.axis_index("s")
    wid = cid * NS + sid

    zero = jnp.zeros((LANES,), f32)

    def zbody(i, carry):
      for k in range(CHUNK // LANES):
        deg2_v[i, pl.ds(k * LANES, LANES)] = zero
        deg_v[pl.ds(i * CHUNK + k * LANES, LANES)] = zero
      return carry
    lax.fori_loop(0, nrows, zbody, 0)

    @pl.when(sid == 0)
    def _():
      pltpu.sync_copy(deg2_v, deg_sh)

    def ibody(k, carry):
      ident_v[pl.ds(k * LANES, LANES)] = (
          lax.iota(jnp.int32, LANES) + k * LANES)
      return carry
    lax.fori_loop(0, nrows // LANES, ibody, 0)

    pltpu.sync_copy(dst_hbm.at[pl.ds(wid * cpw, cpw)], idx_v)

    ones = jnp.ones((LANES,), f32)

    def ebody(i, carry):
      for k in range(CHUNK // LANES):
        d = idx_v[i, pl.ds(k * LANES, LANES)]
        plsc.addupdate_scatter(deg_v, [d], ones)
      return carry
    lax.fori_loop(0, cpw, ebody, 0)

    def sbody(i, carry):
      for k in range(CHUNK // LANES):
        deg2_v[i, pl.ds(k * LANES, LANES)] = (
            deg_v[pl.ds(i * CHUNK + k * LANES, LANES)])
      return carry
    lax.fori_loop(0, nrows, sbody, 0)

    plsc.subcore_barrier()
    pltpu.sync_copy(deg2_v, deg_sh.at[ident_v], add=True)
    plsc.subcore_barrier()

    @pl.when(sid < nrows // 8)
    def _():
      base = sid * 8
      pltpu.sync_copy(deg_sh.at[pl.ds(base, 8)], deg2_v.at[pl.ds(0, 8)])
      pltpu.sync_copy(deg2_v.at[pl.ds(0, 8)], out_hbm.at[cid, pl.ds(base, 8)])

  return deg_kernel


def _sc_propagate(src2d, dst2d, u, npad, cpw, h):
  rows_per_tile = npad // NS
  assert rows_per_tile % CHUNK == 0 or rows_per_tile < CHUNK
  ncopy = rows_per_tile // CHUNK

  @functools.partial(
      pl.kernel,
      out_type=jax.ShapeDtypeStruct((NC, npad, h), f32),
      mesh=_mesh(),
      compiler_params=pltpu.CompilerParams(needs_layout_passes=False,
                                           use_tc_tiling_on_sc=False),
      scratch_types=[
          pltpu.VMEM((cpw, CHUNK), jnp.int32),
          pltpu.VMEM((cpw, CHUNK), jnp.int32),
          [pltpu.VMEM((CHUNK, h), f32) for _ in range(NBUF)],
          pltpu.VMEM_SHARED((npad, h), f32),
          [pltpu.SemaphoreType.DMA for _ in range(NBUF)],
      ],
  )
  def prop_kernel(src_hbm, dst_hbm, u_hbm, out_hbm, src_v, dst_v, rows,
                  acc_sh, sems):
    cid = lax.axis_index("c")
    sid = lax.axis_index("s")
    wid = cid * NS + sid

    zero = jnp.zeros((LANES,), f32)

    def zbody(i, carry):
      for k in range(h // LANES):
        rows[0][i, pl.ds(k * LANES, LANES)] = zero
      return carry
    lax.fori_loop(0, CHUNK, zbody, 0)

    base = sid * rows_per_tile
    for k in range(ncopy):
      pltpu.sync_copy(rows[0], acc_sh.at[pl.ds(base + k * CHUNK, CHUNK)])

    pltpu.sync_copy(src_hbm.at[pl.ds(wid * cpw, cpw)], src_v)
    pltpu.sync_copy(dst_hbm.at[pl.ds(wid * cpw, cpw)], dst_v)

    plsc.subcore_barrier()

    for b in range(NBUF):
      pltpu.async_copy(u_hbm.at[src_v.at[b]], rows[b], sems[b])

    def ring_body(i, carry):
      j0 = i * NBUF
      for b in range(NBUF):
        j = j0 + b
        pltpu.make_async_copy(u_hbm.at[src_v.at[j]], rows[b],
                              sems[b]).wait()
        pltpu.sync_copy(rows[b], acc_sh.at[dst_v.at[j]], add=True)
        @pl.when(j + NBUF < cpw)
        def _():
          pltpu.async_copy(u_hbm.at[src_v.at[j + NBUF]], rows[b], sems[b])
      return carry
    lax.fori_loop(0, cpw // NBUF, ring_body, 0)

    plsc.subcore_barrier()

    for k in range(ncopy):
      pltpu.sync_copy(acc_sh.at[pl.ds(base + k * CHUNK, CHUNK)], rows[0])
      pltpu.sync_copy(rows[0], out_hbm.at[cid, pl.ds(base + k * CHUNK, CHUNK)])

  return prop_kernel(src2d, dst2d, u)


def _tc_pre(degsum, x, W1, n, h):
  def body(deg_ref, x_ref, w_ref, u_ref, dis_ref):
    d = jnp.maximum(deg_ref[...], 1.0)
    dis = lax.rsqrt(d)
    dis = dis * (1.5 - 0.5 * d * dis * dis)
    z = jnp.dot(x_ref[...], w_ref[...], preferred_element_type=f32,
                precision=lax.Precision.HIGHEST)
    u_ref[...] = dis * z
    dis_ref[...] = dis

  return pl.pallas_call(
      body,
      out_shape=(jax.ShapeDtypeStruct((n, h), f32),
                 jax.ShapeDtypeStruct((n, 1), f32)),
  )(degsum, x, W1)


def _tc_mid(S, u, dis, b, W, n, h):
  def body(S_ref, u_ref, dis_ref, b_ref, w_ref, out_ref):
    s = S_ref[0, :n, :] + S_ref[1, :n, :] + u_ref[...]
    hh = jnp.maximum(dis_ref[...] * s + b_ref[...], 0.0)
    out_ref[...] = dis_ref[...] * jnp.dot(
        hh, w_ref[...], preferred_element_type=f32,
        precision=lax.Precision.HIGHEST)

  return pl.pallas_call(
      body,
      out_shape=jax.ShapeDtypeStruct((n, h), f32),
  )(S, u, dis, b, W)


def _tc_final(S, u, dis, b3, Wg, bg, Wn1, bn1, Wn2, bn2, n, h):
  def body(S_ref, u_ref, dis_ref, b_ref, wg_ref, bg_ref, wn1_ref, bn1_ref,
           wn2_ref, bn2_ref, glog_ref, nlog_ref):
    s = S_ref[0, :n, :] + S_ref[1, :n, :] + u_ref[...]
    emb = jnp.maximum(dis_ref[...] * s + b_ref[...], 0.0)
    pooled = jnp.sum(emb, axis=0, keepdims=True) * (1.0 / n)
    glog_ref[...] = jnp.dot(pooled, wg_ref[...], preferred_element_type=f32,
                            precision=lax.Precision.HIGHEST) + bg_ref[...]
    nh = jnp.maximum(
        jnp.dot(emb, wn1_ref[...], preferred_element_type=f32,
                precision=lax.Precision.HIGHEST) + bn1_ref[...], 0.0)
    nlog_ref[...] = jnp.dot(nh, wn2_ref[...], preferred_element_type=f32,
                            precision=lax.Precision.HIGHEST) + bn2_ref[...]

  return pl.pallas_call(
      body,
      out_shape=(jax.ShapeDtypeStruct((1, 2), f32),
                 jax.ShapeDtypeStruct((n, 2), f32)),
  )(S, u, dis, b3, Wg, bg, Wn1, bn1, Wn2, bn2)


def kernel(x, edge_index, batch, W1, b1, W2, b2, W3, b3, Wg, bg, Wn1, bn1,
           Wn2, bn2):
  n, f_in = x.shape
  h = W1.shape[1]
  e = edge_index.shape[1]

  npad = -(-n // (NS * CHUNK)) * (NS * CHUNK)
  cpw = -(-e // (NW * CHUNK))
  cpw = -(-cpw // 8) * 8
  epad = NW * cpw * CHUNK

  trash = npad - 8
  src_p = jnp.concatenate(
      [edge_index[0], jnp.zeros((epad - e,), jnp.int32)]).reshape(-1, CHUNK)
  dst_p = jnp.concatenate(
      [edge_index[1], jnp.full((epad - e,), trash, jnp.int32)]).reshape(
          -1, CHUNK)

  degp = _sc_degree(dst_p, npad, cpw)(dst_p)
  degsum = (degp[0] + degp[1]).reshape(npad, 1)[:n] + 1.0

  u1, dis = _tc_pre(degsum, x, W1, n, h)
  S1 = _sc_propagate(src_p, dst_p, u1, npad, cpw, h)
  u2 = _tc_mid(S1, u1, dis, b1.reshape(1, h), W2, n, h)
  S2 = _sc_propagate(src_p, dst_p, u2, npad, cpw, h)
  u3 = _tc_mid(S2, u2, dis, b2.reshape(1, h), W3, n, h)
  S3 = _sc_propagate(src_p, dst_p, u3, npad, cpw, h)

  return _tc_final(S3, u3, dis, b3.reshape(1, h), Wg, bg.reshape(1, 2),
                   Wn1, bn1.reshape(1, h // 2), Wn2, bn2.reshape(1, 2), n, h)
